# baseline (speedup 1.0000x reference)
import functools

import jax
import jax.numpy as jnp
from jax import lax
from jax.experimental import pallas as pl
from jax.experimental.pallas import tpu as pltpu
from jax.experimental.pallas import tpu_sc as plsc

E = 8
D = 1024
V = 4096
N = 2048
T = 128
NT = N // T + E
NP = NT * T
NTM = 32
NV = 8
MAXQ = N // T
VT = V // NV

NC, NS, L = 2, 16, 16
_MESH = plsc.VectorSubcoreMesh(
    core_axis_name="c", subcore_axis_name="s", num_cores=NC, num_subcores=NS)

TPW = N // NS
RPW = N // (NC * NS)
GCH = 16


def _route_body(addr_hbm, dout_hbm, group_hbm, active_hbm, row_hbm, pt_hbm, q_hbm,
                cntd_hbm,
                addr_v, dout_v, cnt_v, tbl_v, grp_v, act_v, row_v, pt_v, q_v):
    c = lax.axis_index("c")
    s = lax.axis_index("s")
    lanes = lax.iota(jnp.int32, L)

    @pl.when(c == 0)
    def _():
        pltpu.sync_copy(addr_hbm.at[pl.ds(s * TPW, TPW)], addr_v)

        def p1(k, cnt):
            v = addr_v[pl.ds(k * L, L)] & (E - 1)

            def pe(e, cnt):
                m = v == e
                ce = plsc.all_reduce_population_count(m)
                return cnt + jnp.where(lanes == e, ce, 0)

            return lax.fori_loop(0, E, pe, cnt)

        cnt = lax.fori_loop(0, TPW // L, p1, jnp.zeros((L,), jnp.int32))
        cnt_v[...] = cnt
        pltpu.sync_copy(cnt_v, cntd_hbm.at[s])

    plsc.subcore_barrier()

    @pl.when(c == 0)
    def _():
        pltpu.sync_copy(cntd_hbm, tbl_v)

        tot = jnp.zeros((L,), jnp.int32)
        myb = jnp.zeros((L,), jnp.int32)
        for k in range(NS):
            row = tbl_v[k]
            tot = tot + row
            myb = myb + row * (k < s).astype(jnp.int32)

        q = (tot + (T - 1)) >> 7
        s_excl = plsc.cumsum(q) - q
        base = s_excl * T + myb

        def p2(k, base):
            v = addr_v[pl.ds(k * L, L)] & (E - 1)

            def pe(e, carry):
                d, base = carry
                m = v == e
                r = plsc.cumsum(m.astype(jnp.int32)) - 1
                be = jnp.sum(jnp.where(lanes == e, base, 0))
                d = jnp.where(m, be + r, d)
                ce = plsc.all_reduce_population_count(m)
                base = base + jnp.where(lanes == e, ce, 0)
                return d, base

            d, base = lax.fori_loop(0, E, pe,
                                    (jnp.zeros((L,), jnp.int32), base))
            dout_v[k] = d
            return base

        lax.fori_loop(0, TPW // L, p2, base)
        pltpu.sync_copy(dout_v, dout_hbm.at[pl.ds(s * (TPW // L), TPW // L)])

        @pl.when(s == 0)
        def _():
            used = jnp.sum(q)
            glast = jnp.max(jnp.where(q > 0, lanes, 0))

            def tile_chunk(k, _):
                tid = lax.iota(jnp.int32, L) + k * L

                def ge(e, g):
                    ende = jnp.sum(jnp.where(lanes == e, s_excl + q, 0))
                    return g + (tid >= ende).astype(jnp.int32)

                g = lax.fori_loop(0, E, ge, jnp.zeros((L,), jnp.int32))
                act = tid < used
                grp_v[pl.ds(k * L, L)] = jnp.where(act, g, glast)
                act_v[pl.ds(k * L, L)] = act.astype(jnp.int32)
                row_v[pl.ds(k * L, L)] = jnp.where(act, tid, used - 1)
                return 0

            lax.fori_loop(0, NTM // L, tile_chunk, 0)
            pt_v[...] = s_excl
            q_v[...] = q
            pltpu.sync_copy(grp_v, group_hbm)
            pltpu.sync_copy(act_v, active_hbm)
            pltpu.sync_copy(row_v, row_hbm)
            pltpu.sync_copy(pt_v, pt_hbm)
            pltpu.sync_copy(q_v, q_hbm)


_route = functools.partial(
    pl.kernel,
    out_type=(jax.ShapeDtypeStruct((N // L, L), jnp.int32),
              jax.ShapeDtypeStruct((NTM,), jnp.int32),
              jax.ShapeDtypeStruct((NTM,), jnp.int32),
              jax.ShapeDtypeStruct((NTM,), jnp.int32),
              jax.ShapeDtypeStruct((L,), jnp.int32),
              jax.ShapeDtypeStruct((L,), jnp.int32),
              jax.ShapeDtypeStruct((NS, L), jnp.int32)),
    mesh=_MESH,
    compiler_params=pltpu.CompilerParams(needs_layout_passes=False),
    scratch_types=[
        pltpu.VMEM((TPW,), jnp.int32),
        pltpu.VMEM((TPW // L, L), jnp.int32),
        pltpu.VMEM((L,), jnp.int32),
        pltpu.VMEM((NS, L), jnp.int32),
        pltpu.VMEM((NTM,), jnp.int32),
        pltpu.VMEM((NTM,), jnp.int32),
        pltpu.VMEM((NTM,), jnp.int32),
        pltpu.VMEM((L,), jnp.int32),
        pltpu.VMEM((L,), jnp.int32),
    ],
)(_route_body)


def _scatter_body(x_hbm, dout_hbm, xp_hbm, idx_v, buf0, buf1, sem0, sem1):
    c = lax.axis_index("c")
    s = lax.axis_index("s")
    wid = s * NC + c
    nch = RPW // L
    base = wid * nch
    pltpu.sync_copy(dout_hbm.at[pl.ds(base, nch)], idx_v)

    bufs = (buf0, buf1)
    sems = (sem0, sem1)
    cps = [None] * nch
    cps[0] = pltpu.async_copy(x_hbm.at[pl.ds(base * L, L)], bufs[0], sems[0])
    for k in range(nch):
        if k + 1 < nch:
            cps[k + 1] = pltpu.async_copy(
                x_hbm.at[pl.ds((base + k + 1) * L, L)],
                bufs[(k + 1) % 2], sems[(k + 1) % 2])
        cps[k].wait()
        pltpu.sync_copy(bufs[k % 2], xp_hbm.at[idx_v.at[k]])


_scatter = functools.partial(
    pl.kernel,
    out_type=jax.ShapeDtypeStruct((NP, D), jnp.float32),
    mesh=_MESH,
    compiler_params=pltpu.CompilerParams(needs_layout_passes=False),
    scratch_types=[
        pltpu.VMEM((RPW // L, L), jnp.int32),
        pltpu.VMEM((L, D), jnp.float32),
        pltpu.VMEM((L, D), jnp.float32),
        pltpu.SemaphoreType.DMA,
        pltpu.SemaphoreType.DMA,
    ],
)(_scatter_body)


def _mm_body(pt_ref, q_ref, xp_ref, w_ref, b_ref, o_ref):
    e = pl.program_id(1)
    qe = q_ref[e]
    sbase = pt_ref[e] * T

    for kq in range(MAXQ):
        @pl.when(kq < qe)
        def _():
            rows = pl.ds(sbase + kq * T, T)
            xt = xp_ref[rows, :].astype(jnp.bfloat16)
            acc = lax.dot_general(xt, w_ref[0].astype(jnp.bfloat16),
                                  (((1,), (1,)), ((), ())),
                                  preferred_element_type=jnp.float32)
            o_ref[rows, :] = acc + b_ref[0, 0]


def _grouped_matmul(pt, q, xp, w, b):
    grid_spec = pltpu.PrefetchScalarGridSpec(
        num_scalar_prefetch=2,
        grid=(NV, E),
        in_specs=[
            pl.BlockSpec((NP, D), lambda j, e, p, qq: (0, 0)),
            pl.BlockSpec((1, VT, D), lambda j, e, p, qq: (e, j, 0)),
            pl.BlockSpec((1, 1, VT), lambda j, e, p, qq: (e, 0, j)),
        ],
        out_specs=pl.BlockSpec((NP, VT), lambda j, e, p, qq: (0, j)),
    )
    return pl.pallas_call(
        _mm_body,
        grid_spec=grid_spec,
        out_shape=jax.ShapeDtypeStruct((NP, V), jnp.float32),
        compiler_params=pltpu.CompilerParams(
            dimension_semantics=("arbitrary", "arbitrary")),
    )(pt, q, xp, w, b.reshape(E, 1, V))


def _gather_body(outp_hbm, dout_hbm, out_hbm, idx_v, buf, sem):
    c = lax.axis_index("c")
    s = lax.axis_index("s")
    wid = s * NC + c
    nch = RPW // GCH
    base = wid * nch
    pltpu.sync_copy(dout_hbm.at[pl.ds(base, nch)], idx_v)
    for k in range(nch):
        pltpu.async_copy(outp_hbm.at[idx_v.at[k]], buf, sem).wait()
        pltpu.sync_copy(buf, out_hbm.at[pl.ds((base + k) * GCH, GCH)])


_gather = functools.partial(
    pl.kernel,
    out_type=jax.ShapeDtypeStruct((N, V), jnp.float32),
    mesh=_MESH,
    compiler_params=pltpu.CompilerParams(needs_layout_passes=False),
    scratch_types=[
        pltpu.VMEM((RPW // GCH, GCH), jnp.int32),
        pltpu.VMEM((GCH, V), jnp.float32),
        pltpu.SemaphoreType.DMA,
    ],
)(_gather_body)


def kernel(x, pointer_addresses, W, b):
    addr = pointer_addresses.astype(jnp.int32)
    dout, group, active, rowm, pt, q, _ = _route(addr)
    xp = _scatter(x, dout)
    outp = _grouped_matmul(pt, q, xp, W, b)
    return _gather(outp, dout)


# speedup vs baseline: 1.1324x; 1.1324x over previous
import functools

import jax
import jax.numpy as jnp
from jax import lax
from jax.experimental import pallas as pl
from jax.experimental.pallas import tpu as pltpu
from jax.experimental.pallas import tpu_sc as plsc

E = 8
D = 1024
V = 4096
N = 2048
T = 128
NT = N // T + E
NP = NT * T
NTM = 32
NV = 4
MAXQ = N // T
VT = V // NV

NC, NS, L = 2, 16, 16
_MESH = plsc.VectorSubcoreMesh(
    core_axis_name="c", subcore_axis_name="s", num_cores=NC, num_subcores=NS)

TPW = N // NS
RPW = N // (NC * NS)
GCH = 16


def _route_body(addr_hbm, dout_hbm, group_hbm, active_hbm, row_hbm, pt_hbm, q_hbm,
                cntd_hbm,
                addr_v, dout_v, cnt_v, tbl_v, grp_v, act_v, row_v, pt_v, q_v):
    c = lax.axis_index("c")
    s = lax.axis_index("s")
    lanes = lax.iota(jnp.int32, L)

    @pl.when(c == 0)
    def _():
        pltpu.sync_copy(addr_hbm.at[pl.ds(s * TPW, TPW)], addr_v)

        def p1(k, cnt):
            v = addr_v[pl.ds(k * L, L)] & (E - 1)

            def pe(e, cnt):
                m = v == e
                ce = plsc.all_reduce_population_count(m)
                return cnt + jnp.where(lanes == e, ce, 0)

            return lax.fori_loop(0, E, pe, cnt)

        cnt = lax.fori_loop(0, TPW // L, p1, jnp.zeros((L,), jnp.int32))
        cnt_v[...] = cnt
        pltpu.sync_copy(cnt_v, cntd_hbm.at[s])

    plsc.subcore_barrier()

    @pl.when(c == 0)
    def _():
        pltpu.sync_copy(cntd_hbm, tbl_v)

        tot = jnp.zeros((L,), jnp.int32)
        myb = jnp.zeros((L,), jnp.int32)
        for k in range(NS):
            row = tbl_v[k]
            tot = tot + row
            myb = myb + row * (k < s).astype(jnp.int32)

        q = (tot + (T - 1)) >> 7
        s_excl = plsc.cumsum(q) - q
        base = s_excl * T + myb

        def p2(k, base):
            v = addr_v[pl.ds(k * L, L)] & (E - 1)

            def pe(e, carry):
                d, base = carry
                m = v == e
                r = plsc.cumsum(m.astype(jnp.int32)) - 1
                be = jnp.sum(jnp.where(lanes == e, base, 0))
                d = jnp.where(m, be + r, d)
                ce = plsc.all_reduce_population_count(m)
                base = base + jnp.where(lanes == e, ce, 0)
                return d, base

            d, base = lax.fori_loop(0, E, pe,
                                    (jnp.zeros((L,), jnp.int32), base))
            dout_v[k] = d
            return base

        lax.fori_loop(0, TPW // L, p2, base)
        pltpu.sync_copy(dout_v, dout_hbm.at[pl.ds(s * (TPW // L), TPW // L)])

        @pl.when(s == 0)
        def _():
            used = jnp.sum(q)
            glast = jnp.max(jnp.where(q > 0, lanes, 0))

            def tile_chunk(k, _):
                tid = lax.iota(jnp.int32, L) + k * L

                def ge(e, g):
                    ende = jnp.sum(jnp.where(lanes == e, s_excl + q, 0))
                    return g + (tid >= ende).astype(jnp.int32)

                g = lax.fori_loop(0, E, ge, jnp.zeros((L,), jnp.int32))
                act = tid < used
                grp_v[pl.ds(k * L, L)] = jnp.where(act, g, glast)
                act_v[pl.ds(k * L, L)] = act.astype(jnp.int32)
                row_v[pl.ds(k * L, L)] = jnp.where(act, tid, used - 1)
                return 0

            lax.fori_loop(0, NTM // L, tile_chunk, 0)
            pt_v[...] = s_excl
            q_v[...] = q
            pltpu.sync_copy(grp_v, group_hbm)
            pltpu.sync_copy(act_v, active_hbm)
            pltpu.sync_copy(row_v, row_hbm)
            pltpu.sync_copy(pt_v, pt_hbm)
            pltpu.sync_copy(q_v, q_hbm)


_route = functools.partial(
    pl.kernel,
    out_type=(jax.ShapeDtypeStruct((N // L, L), jnp.int32),
              jax.ShapeDtypeStruct((NTM,), jnp.int32),
              jax.ShapeDtypeStruct((NTM,), jnp.int32),
              jax.ShapeDtypeStruct((NTM,), jnp.int32),
              jax.ShapeDtypeStruct((L,), jnp.int32),
              jax.ShapeDtypeStruct((L,), jnp.int32),
              jax.ShapeDtypeStruct((NS, L), jnp.int32)),
    mesh=_MESH,
    compiler_params=pltpu.CompilerParams(needs_layout_passes=False),
    scratch_types=[
        pltpu.VMEM((TPW,), jnp.int32),
        pltpu.VMEM((TPW // L, L), jnp.int32),
        pltpu.VMEM((L,), jnp.int32),
        pltpu.VMEM((NS, L), jnp.int32),
        pltpu.VMEM((NTM,), jnp.int32),
        pltpu.VMEM((NTM,), jnp.int32),
        pltpu.VMEM((NTM,), jnp.int32),
        pltpu.VMEM((L,), jnp.int32),
        pltpu.VMEM((L,), jnp.int32),
    ],
)(_route_body)


def _scatter_body(x_hbm, dout_hbm, xp_hbm, idx_v, buf0, buf1, sem0, sem1):
    c = lax.axis_index("c")
    s = lax.axis_index("s")
    wid = s * NC + c
    nch = RPW // L
    base = wid * nch
    pltpu.sync_copy(dout_hbm.at[pl.ds(base, nch)], idx_v)

    bufs = (buf0, buf1)
    sems = (sem0, sem1)
    cps = [None] * nch
    cps[0] = pltpu.async_copy(x_hbm.at[pl.ds(base * L, L)], bufs[0], sems[0])
    for k in range(nch):
        if k + 1 < nch:
            cps[k + 1] = pltpu.async_copy(
                x_hbm.at[pl.ds((base + k + 1) * L, L)],
                bufs[(k + 1) % 2], sems[(k + 1) % 2])
        cps[k].wait()
        pltpu.sync_copy(bufs[k % 2], xp_hbm.at[idx_v.at[k]])


_scatter = functools.partial(
    pl.kernel,
    out_type=jax.ShapeDtypeStruct((NP, D), jnp.float32),
    mesh=_MESH,
    compiler_params=pltpu.CompilerParams(needs_layout_passes=False),
    scratch_types=[
        pltpu.VMEM((RPW // L, L), jnp.int32),
        pltpu.VMEM((L, D), jnp.float32),
        pltpu.VMEM((L, D), jnp.float32),
        pltpu.SemaphoreType.DMA,
        pltpu.SemaphoreType.DMA,
    ],
)(_scatter_body)


def _mm_body(pt_ref, q_ref, xp_ref, w_ref, b_ref, o_ref):
    e = pl.program_id(1)
    qe = q_ref[e]
    sbase = pt_ref[e] * T

    for kq in range(MAXQ):
        @pl.when(kq < qe)
        def _():
            rows = pl.ds(sbase + kq * T, T)
            xt = xp_ref[rows, :].astype(jnp.bfloat16)
            acc = lax.dot_general(xt, w_ref[0].astype(jnp.bfloat16),
                                  (((1,), (1,)), ((), ())),
                                  preferred_element_type=jnp.float32)
            o_ref[rows, :] = acc + b_ref[0, 0]


def _grouped_matmul(pt, q, xp, w, b):
    grid_spec = pltpu.PrefetchScalarGridSpec(
        num_scalar_prefetch=2,
        grid=(NV, E),
        in_specs=[
            pl.BlockSpec((NP, D), lambda j, e, p, qq: (0, 0)),
            pl.BlockSpec((1, VT, D), lambda j, e, p, qq: (e, j, 0)),
            pl.BlockSpec((1, 1, VT), lambda j, e, p, qq: (e, 0, j)),
        ],
        out_specs=pl.BlockSpec((NP, VT), lambda j, e, p, qq: (0, j)),
    )
    return pl.pallas_call(
        _mm_body,
        grid_spec=grid_spec,
        out_shape=jax.ShapeDtypeStruct((NP, V), jnp.float32),
        compiler_params=pltpu.CompilerParams(
            dimension_semantics=("arbitrary", "arbitrary")),
    )(pt, q, xp, w, b.reshape(E, 1, V))


def _gather_body(outp_hbm, dout_hbm, out_hbm, idx_v, buf, sem):
    c = lax.axis_index("c")
    s = lax.axis_index("s")
    wid = s * NC + c
    nch = RPW // GCH
    base = wid * nch
    pltpu.sync_copy(dout_hbm.at[pl.ds(base, nch)], idx_v)
    for k in range(nch):
        pltpu.async_copy(outp_hbm.at[idx_v.at[k]], buf, sem).wait()
        pltpu.sync_copy(buf, out_hbm.at[pl.ds((base + k) * GCH, GCH)])


_gather = functools.partial(
    pl.kernel,
    out_type=jax.ShapeDtypeStruct((N, V), jnp.float32),
    mesh=_MESH,
    compiler_params=pltpu.CompilerParams(needs_layout_passes=False),
    scratch_types=[
        pltpu.VMEM((RPW // GCH, GCH), jnp.int32),
        pltpu.VMEM((GCH, V), jnp.float32),
        pltpu.SemaphoreType.DMA,
    ],
)(_gather_body)


def kernel(x, pointer_addresses, W, b):
    addr = pointer_addresses.astype(jnp.int32)
    dout, group, active, rowm, pt, q, _ = _route(addr)
    xp = _scatter(x, dout)
    outp = _grouped_matmul(pt, q, xp, W, b)
    return _gather(outp, dout)
